# baseline (speedup 1.0000x reference)
import functools

import jax
import jax.numpy as jnp
from jax import lax
from jax.experimental import pallas as pl
from jax.experimental.pallas import tpu as pltpu
from jax.experimental.pallas import tpu_sc as plsc

_NUM_TRAIN = 100000
_BIT = 64
_NCLS = 100
_B = 256
_ALPHA = 0.1

_NBLK = 8192
_NB = 13
_NPAD = _NB * _NBLK
_LOG2E = 1.4426950408889634
_LN2 = 0.6931471805599453

_NW = 32
_SEG = 3328


def _sc_build_mask(ind):
    mesh = plsc.VectorSubcoreMesh(core_axis_name="c", subcore_axis_name="s")

    @functools.partial(
        pl.kernel,
        out_type=jax.ShapeDtypeStruct((_NW * _SEG,), jnp.float32),
        mesh=mesh,
        compiler_params=pltpu.CompilerParams(needs_layout_passes=False),
        scratch_types=[
            pltpu.VMEM((_SEG,), jnp.float32),
            pltpu.VMEM((_B,), jnp.int32),
        ],
    )
    def k(ind_hbm, out_hbm, seg_v, ind_v):
        wid = lax.axis_index("s") * 2 + lax.axis_index("c")
        base = wid * _SEG
        lane = lax.iota(jnp.int32, 16)

        def fill(t, carry):
            for j in range(4):
                off = t * 64 + j * 16
                pos = base + off + lane
                seg_v[pl.ds(off, 16)] = jnp.where(pos < _NUM_TRAIN, 1.0, 0.0)
            return carry

        lax.fori_loop(0, _SEG // 64, fill, 0)
        pltpu.sync_copy(ind_hbm, ind_v)
        zeros = jnp.zeros((16,), jnp.float32)
        for t in range(_B // 16):
            idx = ind_v[pl.ds(t * 16, 16)]
            loc = idx - base
            inb = (loc >= 0) & (loc < _SEG)
            locc = jnp.clip(loc, 0, _SEG - 1)
            plsc.store_scatter(seg_v, [locc], zeros, mask=inb)
        pltpu.sync_copy(seg_v, out_hbm.at[pl.ds(base, _SEG)])

    return k(ind)


def _loss_terms2(ip2, sd):
    return jnp.log2(1.0 + jnp.exp2(ip2)) - jnp.where(sd > 0.0, ip2, 0.0)


def _tc_body(us_ref, uT_ref, y_ref, indr_ref, indc_ref, olT_ref, UT_ref,
             YT_ref, w_ref, out_ref):
    i = pl.program_id(0)
    us = us_ref[...]
    y = y_ref[...]
    dn = (((1,), (1,)), ((), ()))
    dnk = (((1,), (0,)), ((), ()))
    bf = jnp.bfloat16
    ip2 = lax.dot_general(us.astype(bf), UT_ref[...].astype(bf), dnk,
                          preferred_element_type=jnp.float32)
    sd = lax.dot_general(y.astype(bf), YT_ref[...].astype(bf), dnk,
                         preferred_element_type=jnp.float32)
    g = jnp.log2(1.0 + jnp.exp2(ip2)) - jnp.where(sd > 0.0, ip2, 0.0)
    col = lax.broadcasted_iota(jnp.int32, (1, _NBLK), 1) + i * _NBLK
    g = jnp.where(col < _NUM_TRAIN, g, 0.0)
    rows = lax.dot_general(jnp.ones((1, _B), jnp.float32), g,
                           (((1,), (0,)), ((), ())),
                           preferred_element_type=jnp.float32)
    w = jnp.reshape(w_ref[...], (1, _NBLK))
    bsum = jnp.sum(rows * w, keepdims=True)

    @pl.when(i == 0)
    def _():
        out_ref[...] = bsum

    @pl.when(i != 0)
    def _():
        out_ref[...] += bsum

    @pl.when(i == _NB - 1)
    def _():
        uT = uT_ref[...]
        ipn2 = lax.dot_general(us, uT, dnk,
                               preferred_element_type=jnp.float32)
        sdn = lax.dot_general(y, y, dn, preferred_element_type=jnp.float32)
        gn = _loss_terms2(ipn2, sdn)
        eq = indc_ref[...] == indr_ref[...]
        ia = lax.broadcasted_iota(jnp.int32, (_B, _B), 0)
        ib = lax.broadcasted_iota(jnp.int32, (_B, _B), 1)
        dup = jnp.where(eq & (ia > ib), 1.0, 0.0)
        vf = 1.0 - jnp.max(dup, axis=0, keepdims=True)
        new_sum = jnp.sum(gn * vf, keepdims=True)
        olT = olT_ref[...]
        cr = jnp.sum(jnp.maximum(olT, 0.0) - olT * y.T
                     + jnp.log1p(jnp.exp(-jnp.abs(olT))),
                     keepdims=True) / (_B * _NCLS)
        qt = jnp.sum((uT - jnp.sign(uT)) ** 2, keepdims=True) * (
            _ALPHA / (_B * _BIT))
        lik = (out_ref[...] + new_sum) * (_LN2 / (_B * _NUM_TRAIN))
        out_ref[...] = 0.2 * (lik + qt) + 0.8 * cr


def kernel(u, y, ind, out_label, U, Y):
    w_pad = _sc_build_mask(ind)
    w3 = w_pad.reshape(_NB, 1, _NBLK)
    indf = ind.astype(jnp.float32)
    res = pl.pallas_call(
        _tc_body,
        grid=(_NB,),
        in_specs=[
            pl.BlockSpec((_B, _BIT), lambda i: (0, 0)),
            pl.BlockSpec((_BIT, _B), lambda i: (0, 0)),
            pl.BlockSpec((_B, _NCLS), lambda i: (0, 0)),
            pl.BlockSpec((1, _B), lambda i: (0, 0)),
            pl.BlockSpec((_B, 1), lambda i: (0, 0)),
            pl.BlockSpec((_NCLS, _B), lambda i: (0, 0)),
            pl.BlockSpec((_BIT, _NBLK), lambda i: (0, i)),
            pl.BlockSpec((_NCLS, _NBLK), lambda i: (0, i)),
            pl.BlockSpec((1, 1, _NBLK), lambda i: (i, 0, 0)),
        ],
        out_specs=pl.BlockSpec((1, 1), lambda i: (0, 0)),
        out_shape=jax.ShapeDtypeStruct((1, 1), jnp.float32),
        compiler_params=pltpu.CompilerParams(
            dimension_semantics=("arbitrary",)),
    )(u * (0.5 * _LOG2E), u.T, y, indf.reshape(1, _B),
      indf.reshape(_B, 1), out_label.T, U.T, Y.T, w3)
    return res[0, 0]


# speedup vs baseline: 1.0406x; 1.0406x over previous
import functools

import jax
import jax.numpy as jnp
from jax import lax
from jax.experimental import pallas as pl
from jax.experimental.pallas import tpu as pltpu
from jax.experimental.pallas import tpu_sc as plsc

_NUM_TRAIN = 100000
_BIT = 64
_NCLS = 100
_B = 256
_ALPHA = 0.1

_NBLK = 4096
_NB = 25
_NPAD = _NB * _NBLK
_LOG2E = 1.4426950408889634
_LN2 = 0.6931471805599453

_NW = 32
_SEG = 3200


def _sc_build_mask(ind):
    mesh = plsc.VectorSubcoreMesh(core_axis_name="c", subcore_axis_name="s")

    @functools.partial(
        pl.kernel,
        out_type=jax.ShapeDtypeStruct((_NW * _SEG,), jnp.float32),
        mesh=mesh,
        compiler_params=pltpu.CompilerParams(needs_layout_passes=False),
        scratch_types=[
            pltpu.VMEM((_SEG,), jnp.float32),
            pltpu.VMEM((_B,), jnp.int32),
        ],
    )
    def k(ind_hbm, out_hbm, seg_v, ind_v):
        wid = lax.axis_index("s") * 2 + lax.axis_index("c")
        base = wid * _SEG
        lane = lax.iota(jnp.int32, 16)

        def fill(t, carry):
            for j in range(4):
                off = t * 64 + j * 16
                pos = base + off + lane
                seg_v[pl.ds(off, 16)] = jnp.where(pos < _NUM_TRAIN, 1.0, 0.0)
            return carry

        lax.fori_loop(0, _SEG // 64, fill, 0)
        pltpu.sync_copy(ind_hbm, ind_v)
        zeros = jnp.zeros((16,), jnp.float32)
        for t in range(_B // 16):
            idx = ind_v[pl.ds(t * 16, 16)]
            loc = idx - base
            inb = (loc >= 0) & (loc < _SEG)
            locc = jnp.clip(loc, 0, _SEG - 1)
            plsc.store_scatter(seg_v, [locc], zeros, mask=inb)
        pltpu.sync_copy(seg_v, out_hbm.at[pl.ds(base, _SEG)])

    return k(ind)


def _loss_terms2(ip2, sd):
    return jnp.log2(1.0 + jnp.exp2(ip2)) - jnp.where(sd > 0.0, ip2, 0.0)


def _tc_body(us_ref, uT_ref, y_ref, indr_ref, indc_ref, olT_ref, UT_ref,
             YT_ref, w_ref, out_ref):
    i = pl.program_id(0)
    us = us_ref[...]
    y = y_ref[...]
    dn = (((1,), (1,)), ((), ()))
    dnk = (((1,), (0,)), ((), ()))
    bf = jnp.bfloat16
    ip2 = lax.dot_general(us.astype(bf), UT_ref[...].astype(bf), dnk,
                          preferred_element_type=jnp.float32)
    sd = lax.dot_general(y.astype(bf), YT_ref[...].astype(bf), dnk,
                         preferred_element_type=jnp.float32)
    g = jnp.log2(1.0 + jnp.exp2(ip2)) - jnp.where(sd > 0.0, ip2, 0.0)
    col = lax.broadcasted_iota(jnp.int32, (1, _NBLK), 1) + i * _NBLK
    g = jnp.where(col < _NUM_TRAIN, g, 0.0)
    rows = lax.dot_general(jnp.ones((1, _B), jnp.float32), g,
                           (((1,), (0,)), ((), ())),
                           preferred_element_type=jnp.float32)
    w = jnp.reshape(w_ref[...], (1, _NBLK))
    bsum = jnp.sum(rows * w, keepdims=True)

    @pl.when(i == 0)
    def _():
        out_ref[...] = bsum

    @pl.when(i != 0)
    def _():
        out_ref[...] += bsum

    @pl.when(i == _NB - 1)
    def _():
        uT = uT_ref[...]
        ipn2 = lax.dot_general(us, uT, dnk,
                               preferred_element_type=jnp.float32)
        sdn = lax.dot_general(y, y, dn, preferred_element_type=jnp.float32)
        gn = _loss_terms2(ipn2, sdn)
        eq = indc_ref[...] == indr_ref[...]
        ia = lax.broadcasted_iota(jnp.int32, (_B, _B), 0)
        ib = lax.broadcasted_iota(jnp.int32, (_B, _B), 1)
        dup = jnp.where(eq & (ia > ib), 1.0, 0.0)
        vf = 1.0 - jnp.max(dup, axis=0, keepdims=True)
        new_sum = jnp.sum(gn * vf, keepdims=True)
        olT = olT_ref[...]
        cr = jnp.sum(jnp.maximum(olT, 0.0) - olT * y.T
                     + jnp.log1p(jnp.exp(-jnp.abs(olT))),
                     keepdims=True) / (_B * _NCLS)
        qt = jnp.sum((uT - jnp.sign(uT)) ** 2, keepdims=True) * (
            _ALPHA / (_B * _BIT))
        lik = (out_ref[...] + new_sum) * (_LN2 / (_B * _NUM_TRAIN))
        out_ref[...] = 0.2 * (lik + qt) + 0.8 * cr


def kernel(u, y, ind, out_label, U, Y):
    w_pad = _sc_build_mask(ind)
    w3 = w_pad.reshape(_NB, 1, _NBLK)
    indf = ind.astype(jnp.float32)
    res = pl.pallas_call(
        _tc_body,
        grid=(_NB,),
        in_specs=[
            pl.BlockSpec((_B, _BIT), lambda i: (0, 0)),
            pl.BlockSpec((_BIT, _B), lambda i: (0, 0)),
            pl.BlockSpec((_B, _NCLS), lambda i: (0, 0)),
            pl.BlockSpec((1, _B), lambda i: (0, 0)),
            pl.BlockSpec((_B, 1), lambda i: (0, 0)),
            pl.BlockSpec((_NCLS, _B), lambda i: (0, 0)),
            pl.BlockSpec((_BIT, _NBLK), lambda i: (0, i)),
            pl.BlockSpec((_NCLS, _NBLK), lambda i: (0, i)),
            pl.BlockSpec((1, 1, _NBLK), lambda i: (i, 0, 0)),
        ],
        out_specs=pl.BlockSpec((1, 1), lambda i: (0, 0)),
        out_shape=jax.ShapeDtypeStruct((1, 1), jnp.float32),
        compiler_params=pltpu.CompilerParams(
            dimension_semantics=("arbitrary",)),
    )(u * (0.5 * _LOG2E), u.T, y, indf.reshape(1, _B),
      indf.reshape(_B, 1), out_label.T, U.T, Y.T, w3)
    return res[0, 0]
